# baseline (speedup 1.0000x reference)
import functools

import jax
import jax.numpy as jnp
from jax.experimental import pallas as pl
from jax.experimental.pallas import tpu as pltpu

_EPS = 1e-5


def _fused_encoder(x_ref, m_ref, wpre_ref, bpre_ref,
                   w1a_ref, w1b_ref, b1_ref,
                   w2_ref, b2_ref,
                   wout_ref, bout_ref, out_ref, *, n_pts, tile):
    rows = n_pts * tile
    c = x_ref.shape[-1]
    x = x_ref[...].reshape(rows, c)
    m = m_ref[...].reshape(rows, 1)
    h = jnp.maximum(
        jnp.dot(x, wpre_ref[...], preferred_element_type=jnp.float32)
        + bpre_ref[...], 0.0) * m
    hb = h.astype(jnp.bfloat16)
    pooled = jnp.max(h.reshape(n_pts, tile, -1), axis=0)
    z = jnp.dot(hb, w1a_ref[...], preferred_element_type=jnp.float32)
    z = (z.reshape(n_pts, tile, -1)
         + jnp.dot(pooled.astype(jnp.bfloat16), w1b_ref[...],
                   preferred_element_type=jnp.float32)[None])
    h2 = jnp.maximum(z.reshape(rows, -1) + b1_ref[...], 0.0)
    h2 = jnp.maximum(
        jnp.dot(h2.astype(jnp.bfloat16), w2_ref[...],
                preferred_element_type=jnp.float32)
        + b2_ref[...], 0.0) * m
    poly = jnp.max(h2.reshape(n_pts, tile, -1), axis=0)
    valid = jnp.max(m.reshape(n_pts, tile, 1), axis=0)
    out_ref[...] = (
        jnp.dot(poly.astype(jnp.bfloat16), wout_ref[...],
                preferred_element_type=jnp.float32)
        + bout_ref[...]) * valid


def kernel(polylines, polylines_mask, W_pre, g_pre, b_pre,
           W1, g1, b1, W2, g2, b2, W_out, b_out):
    B, P, N, C = polylines.shape
    H = W_pre.shape[1]
    O = W_out.shape[1]
    M = B * P
    tile = 128
    grid = M // tile

    xt = polylines.astype(jnp.bfloat16).reshape(M, N, C).transpose(1, 0, 2)
    mt = polylines_mask.reshape(M, N).T.astype(jnp.float32)[..., None]

    inv = 1.0 / jnp.sqrt(1.0 + _EPS)
    wpre = (W_pre * (g_pre * inv)).astype(jnp.bfloat16)
    w1 = (W1 * (g1 * inv)).astype(jnp.bfloat16)
    w2 = (W2 * (g2 * inv)).astype(jnp.bfloat16)
    wout = W_out.astype(jnp.bfloat16)

    out = pl.pallas_call(
        functools.partial(_fused_encoder, n_pts=N, tile=tile),
        grid=(grid,),
        in_specs=[
            pl.BlockSpec((N, tile, C), lambda i: (0, i, 0)),
            pl.BlockSpec((N, tile, 1), lambda i: (0, i, 0)),
            pl.BlockSpec((C, H), lambda i: (0, 0)),
            pl.BlockSpec((1, H), lambda i: (0, 0)),
            pl.BlockSpec((H, H), lambda i: (0, 0)),
            pl.BlockSpec((H, H), lambda i: (0, 0)),
            pl.BlockSpec((1, H), lambda i: (0, 0)),
            pl.BlockSpec((H, H), lambda i: (0, 0)),
            pl.BlockSpec((1, H), lambda i: (0, 0)),
            pl.BlockSpec((H, O), lambda i: (0, 0)),
            pl.BlockSpec((1, O), lambda i: (0, 0)),
        ],
        out_specs=pl.BlockSpec((tile, O), lambda i: (i, 0)),
        out_shape=jax.ShapeDtypeStruct((M, O), jnp.float32),
        compiler_params=pltpu.CompilerParams(
            dimension_semantics=("parallel",)),
    )(xt, mt, wpre, b_pre.reshape(1, H),
      w1[:H], w1[H:], b1.reshape(1, H),
      w2, b2.reshape(1, H),
      wout, b_out.reshape(1, O))
    return out.reshape(B, P, O)


# speedup vs baseline: 1.0390x; 1.0390x over previous
import functools

import jax
import jax.numpy as jnp
from jax.experimental import pallas as pl
from jax.experimental.pallas import tpu as pltpu

_EPS = 1e-5


def _fused_encoder(*refs, n_pts, tile):
    x_refs = refs[:n_pts]
    (m_ref, wpre_ref, bpre_ref, w1a_ref, w1b_ref, b1_ref,
     w2_ref, b2_ref, wout_ref, bout_ref, out_ref) = refs[n_pts:]
    rows = n_pts * tile
    m3 = m_ref[...]
    wpre = wpre_ref[...]
    bpre = bpre_ref[...]
    hs = [jnp.maximum(
              jnp.dot(x_refs[n][...], wpre, preferred_element_type=jnp.float32)
              + bpre, 0.0) * m3[:, n:n + 1]
          for n in range(n_pts)]
    pooled = functools.reduce(jnp.maximum, hs)
    h = jnp.stack(hs, axis=0).reshape(rows, -1)
    z = jnp.dot(h, w1a_ref[...], preferred_element_type=jnp.float32)
    z = (z.reshape(n_pts, tile, -1)
         + jnp.dot(pooled, w1b_ref[...], preferred_element_type=jnp.float32)[None])
    h2 = jnp.maximum(z.reshape(rows, -1) + b1_ref[...], 0.0)
    h2 = jnp.maximum(
        jnp.dot(h2, w2_ref[...], preferred_element_type=jnp.float32)
        + b2_ref[...], 0.0)
    h2 = h2.reshape(n_pts, tile, -1)
    poly = functools.reduce(
        jnp.maximum, [h2[n] * m3[:, n:n + 1] for n in range(n_pts)])
    valid = jnp.max(m3, axis=1, keepdims=True)
    out_ref[...] = (
        jnp.dot(poly, wout_ref[...], preferred_element_type=jnp.float32)
        + bout_ref[...]) * valid


def kernel(polylines, polylines_mask, W_pre, g_pre, b_pre,
           W1, g1, b1, W2, g2, b2, W_out, b_out):
    B, P, N, C = polylines.shape
    H = W_pre.shape[1]
    O = W_out.shape[1]
    M = B * P
    tile = 256
    grid = M // tile

    xr = polylines.reshape(M, N, C)
    xs = tuple(xr[:, n, :] for n in range(N))
    mf = polylines_mask.reshape(M, N).astype(jnp.float32)

    inv = 1.0 / jnp.sqrt(1.0 + _EPS)
    wpre = W_pre * (g_pre * inv)
    w1 = W1 * (g1 * inv)
    w2 = W2 * (g2 * inv)

    row_spec = pl.BlockSpec((tile, C), lambda i: (i, 0))
    full = lambda shape: pl.BlockSpec(shape, lambda i: (0, 0))

    out = pl.pallas_call(
        functools.partial(_fused_encoder, n_pts=N, tile=tile),
        grid=(grid,),
        in_specs=[row_spec] * N + [
            pl.BlockSpec((tile, N), lambda i: (i, 0)),
            full((C, H)),
            full((1, H)),
            full((H, H)),
            full((H, H)),
            full((1, H)),
            full((H, H)),
            full((1, H)),
            full((H, O)),
            full((1, O)),
        ],
        out_specs=pl.BlockSpec((tile, O), lambda i: (i, 0)),
        out_shape=jax.ShapeDtypeStruct((M, O), jnp.float32),
        compiler_params=pltpu.CompilerParams(
            dimension_semantics=("parallel",)),
    )(*xs, mf, wpre, b_pre.reshape(1, H),
      w1[:H], w1[H:], b1.reshape(1, H),
      w2, b2.reshape(1, H),
      W_out, b_out.reshape(1, O))
    return out.reshape(B, P, O)


# speedup vs baseline: 1.2321x; 1.1859x over previous
import functools

import jax
import jax.numpy as jnp
from jax.experimental import pallas as pl
from jax.experimental.pallas import tpu as pltpu

_EPS = 1e-5


def _fused_encoder(x_ref, m_ref, wpre_ref, bpre_ref,
                   w1a_ref, w1b_ref, b1_ref,
                   w2_ref, b2_ref, wout_ref, bout_ref, out_ref,
                   *, n_pts, tile):
    rows = n_pts * tile
    x3 = x_ref[...]
    m3 = m_ref[...]
    wpre = wpre_ref[...]
    bpre = bpre_ref[...]
    hs = [jnp.maximum(
              jnp.dot(x3[:, n, :], wpre, preferred_element_type=jnp.float32)
              + bpre, 0.0) * m3[:, n:n + 1]
          for n in range(n_pts)]
    pooled = functools.reduce(jnp.maximum, hs)
    h = jnp.stack(hs, axis=0).reshape(rows, -1)
    z = jnp.dot(h, w1a_ref[...], preferred_element_type=jnp.float32)
    z = (z.reshape(n_pts, tile, -1)
         + jnp.dot(pooled, w1b_ref[...], preferred_element_type=jnp.float32)[None])
    h2 = jnp.maximum(z.reshape(rows, -1) + b1_ref[...], 0.0)
    h2 = jnp.maximum(
        jnp.dot(h2, w2_ref[...], preferred_element_type=jnp.float32)
        + b2_ref[...], 0.0)
    h2 = h2.reshape(n_pts, tile, -1)
    poly = functools.reduce(
        jnp.maximum, [h2[n] * m3[:, n:n + 1] for n in range(n_pts)])
    valid = jnp.max(m3, axis=1, keepdims=True)
    out_ref[...] = (
        jnp.dot(poly, wout_ref[...], preferred_element_type=jnp.float32)
        + bout_ref[...]) * valid


def kernel(polylines, polylines_mask, W_pre, g_pre, b_pre,
           W1, g1, b1, W2, g2, b2, W_out, b_out):
    B, P, N, C = polylines.shape
    H = W_pre.shape[1]
    O = W_out.shape[1]
    M = B * P
    tile = 256
    grid = M // tile

    xr = polylines.reshape(M, N, C)
    mf = polylines_mask.reshape(M, N).astype(jnp.float32)

    inv = 1.0 / jnp.sqrt(1.0 + _EPS)
    wpre = W_pre * (g_pre * inv)
    w1 = W1 * (g1 * inv)
    w2 = W2 * (g2 * inv)

    full = lambda shape: pl.BlockSpec(shape, lambda i: (0, 0))

    out = pl.pallas_call(
        functools.partial(_fused_encoder, n_pts=N, tile=tile),
        grid=(grid,),
        in_specs=[
            pl.BlockSpec((tile, N, C), lambda i: (i, 0, 0)),
            pl.BlockSpec((tile, N), lambda i: (i, 0)),
            full((C, H)),
            full((1, H)),
            full((H, H)),
            full((H, H)),
            full((1, H)),
            full((H, H)),
            full((1, H)),
            full((H, O)),
            full((1, O)),
        ],
        out_specs=pl.BlockSpec((tile, O), lambda i: (i, 0)),
        out_shape=jax.ShapeDtypeStruct((M, O), jnp.float32),
        compiler_params=pltpu.CompilerParams(
            dimension_semantics=("parallel",)),
    )(xr, mf, wpre, b_pre.reshape(1, H),
      w1[:H], w1[H:], b1.reshape(1, H),
      w2, b2.reshape(1, H),
      W_out, b_out.reshape(1, O))
    return out.reshape(B, P, O)


# speedup vs baseline: 1.3405x; 1.0880x over previous
import functools

import jax
import jax.numpy as jnp
from jax.experimental import pallas as pl
from jax.experimental.pallas import tpu as pltpu

_EPS = 1e-5


def _fused_encoder(x_ref, m_ref, wpre_ref, bpre_ref,
                   w1a_ref, w1b_ref, b1_ref,
                   w2_ref, b2_ref, wout_ref, bout_ref, out_ref,
                   *, n_pts, tile):
    rows = n_pts * tile
    x3 = x_ref[...]
    m3 = m_ref[...]
    wpre = wpre_ref[...]
    bpre = bpre_ref[...]
    xt = jnp.swapaxes(x3, 0, 1)
    hs = [jnp.maximum(
              jnp.dot(xt[n], wpre, preferred_element_type=jnp.float32)
              + bpre, 0.0) * m3[:, n:n + 1]
          for n in range(n_pts)]
    pooled = functools.reduce(jnp.maximum, hs)
    t2 = (jnp.dot(pooled, w1b_ref[...], preferred_element_type=jnp.float32)
          + b1_ref[...])
    w1a = w1a_ref[...]
    w2 = w2_ref[...]
    b2 = b2_ref[...]
    poly = None
    for n in range(n_pts):
        zn = jnp.dot(hs[n], w1a, preferred_element_type=jnp.float32) + t2
        h2n = jnp.maximum(zn, 0.0)
        h2n = jnp.maximum(
            jnp.dot(h2n, w2, preferred_element_type=jnp.float32) + b2,
            0.0) * m3[:, n:n + 1]
        poly = h2n if poly is None else jnp.maximum(poly, h2n)
    valid = jnp.max(m3, axis=1, keepdims=True)
    out_ref[...] = (
        jnp.dot(poly, wout_ref[...], preferred_element_type=jnp.float32)
        + bout_ref[...]) * valid


def kernel(polylines, polylines_mask, W_pre, g_pre, b_pre,
           W1, g1, b1, W2, g2, b2, W_out, b_out):
    B, P, N, C = polylines.shape
    H = W_pre.shape[1]
    O = W_out.shape[1]
    M = B * P
    tile = 256
    grid = M // tile

    xr = polylines.reshape(M, N, C)
    mf = polylines_mask.reshape(M, N).astype(jnp.float32)

    inv = 1.0 / jnp.sqrt(1.0 + _EPS)
    wpre = W_pre * (g_pre * inv)
    w1 = W1 * (g1 * inv)
    w2 = W2 * (g2 * inv)

    full = lambda shape: pl.BlockSpec(shape, lambda i: (0, 0))

    out = pl.pallas_call(
        functools.partial(_fused_encoder, n_pts=N, tile=tile),
        grid=(grid,),
        in_specs=[
            pl.BlockSpec((tile, N, C), lambda i: (i, 0, 0)),
            pl.BlockSpec((tile, N), lambda i: (i, 0)),
            full((C, H)),
            full((1, H)),
            full((H, H)),
            full((H, H)),
            full((1, H)),
            full((H, H)),
            full((1, H)),
            full((H, O)),
            full((1, O)),
        ],
        out_specs=pl.BlockSpec((tile, O), lambda i: (i, 0)),
        out_shape=jax.ShapeDtypeStruct((M, O), jnp.float32),
        compiler_params=pltpu.CompilerParams(
            dimension_semantics=("parallel",)),
    )(xr, mf, wpre, b_pre.reshape(1, H),
      w1[:H], w1[H:], b1.reshape(1, H),
      w2, b2.reshape(1, H),
      W_out, b_out.reshape(1, O))
    return out.reshape(B, P, O)


# speedup vs baseline: 1.3576x; 1.0127x over previous
import functools

import jax
import jax.numpy as jnp
from jax.experimental import pallas as pl
from jax.experimental.pallas import tpu as pltpu

_EPS = 1e-5


def _fused_encoder(x_ref, m_ref, wpre_ref, bpre_ref,
                   w1a_ref, w1b_ref, b1_ref,
                   w2_ref, b2_ref, wout_ref, bout_ref, out_ref,
                   *, n_pts, tile):
    rows = n_pts * tile
    x3 = x_ref[...]
    m3 = m_ref[...]
    wpre = wpre_ref[...]
    bpre = bpre_ref[...]
    xt = jnp.swapaxes(x3, 0, 1)
    hs = [jnp.maximum(
              jnp.dot(xt[n], wpre, preferred_element_type=jnp.float32)
              + bpre, 0.0) * m3[:, n:n + 1]
          for n in range(n_pts)]
    pooled = functools.reduce(jnp.maximum, hs)
    t2 = (jnp.dot(pooled, w1b_ref[...], preferred_element_type=jnp.float32)
          + b1_ref[...])
    w1a = w1a_ref[...]
    w2 = w2_ref[...]
    b2 = b2_ref[...]
    poly = None
    for n in range(n_pts):
        zn = jnp.dot(hs[n], w1a, preferred_element_type=jnp.float32) + t2
        h2n = jnp.maximum(zn, 0.0)
        h2n = jnp.maximum(
            jnp.dot(h2n, w2, preferred_element_type=jnp.float32) + b2,
            0.0) * m3[:, n:n + 1]
        poly = h2n if poly is None else jnp.maximum(poly, h2n)
    valid = jnp.max(m3, axis=1, keepdims=True)
    out_ref[...] = (
        jnp.dot(poly, wout_ref[...], preferred_element_type=jnp.float32)
        + bout_ref[...]) * valid


def kernel(polylines, polylines_mask, W_pre, g_pre, b_pre,
           W1, g1, b1, W2, g2, b2, W_out, b_out):
    B, P, N, C = polylines.shape
    H = W_pre.shape[1]
    O = W_out.shape[1]
    M = B * P
    tile = 512
    grid = M // tile

    xr = polylines.reshape(M, N, C)
    mf = polylines_mask.reshape(M, N).astype(jnp.float32)

    inv = 1.0 / jnp.sqrt(1.0 + _EPS)
    wpre = W_pre * (g_pre * inv)
    w1 = W1 * (g1 * inv)
    w2 = W2 * (g2 * inv)

    full = lambda shape: pl.BlockSpec(shape, lambda i: (0, 0))

    out = pl.pallas_call(
        functools.partial(_fused_encoder, n_pts=N, tile=tile),
        grid=(grid,),
        in_specs=[
            pl.BlockSpec((tile, N, C), lambda i: (i, 0, 0)),
            pl.BlockSpec((tile, N), lambda i: (i, 0)),
            full((C, H)),
            full((1, H)),
            full((H, H)),
            full((H, H)),
            full((1, H)),
            full((H, H)),
            full((1, H)),
            full((H, O)),
            full((1, O)),
        ],
        out_specs=pl.BlockSpec((tile, O), lambda i: (i, 0)),
        out_shape=jax.ShapeDtypeStruct((M, O), jnp.float32),
        compiler_params=pltpu.CompilerParams(
            dimension_semantics=("parallel",)),
    )(xr, mf, wpre, b_pre.reshape(1, H),
      w1[:H], w1[H:], b1.reshape(1, H),
      w2, b2.reshape(1, H),
      W_out, b_out.reshape(1, O))
    return out.reshape(B, P, O)
